# baseline (speedup 1.0000x reference)
import functools

import jax
import jax.numpy as jnp
from jax import lax
from jax.experimental import pallas as pl
from jax.experimental.pallas import tpu as pltpu
from jax.experimental.pallas import tpu_sc as plsc

N = 10000
E = 320000
IN_D = 128
HID_D = 128
OUT_D = 64

NC = 2
NS = 16
NW = NC * NS

G = 128
GP = 80
GROUPS_PAD = NW * GP
E_PAD = GROUPS_PAD * G

N_PAD = 10112
RPT = N_PAD // NS
RPT_LAST = N - (NS - 1) * RPT
DUMMY_DST0 = N
N_DUMMY = N_PAD - N


def _sc_degree(dst_groups, zeros, ones):
    mesh = plsc.VectorSubcoreMesh(core_axis_name="c", subcore_axis_name="s")

    @functools.partial(
        pl.kernel,
        mesh=mesh,
        out_type=jax.ShapeDtypeStruct((NC, N, HID_D), jnp.float32),
        scratch_types=[
            pltpu.VMEM((GP, G), jnp.int32),
            pltpu.VMEM((G, HID_D), jnp.float32),
            pltpu.VMEM_SHARED((N_PAD, HID_D), jnp.float32),
        ],
    )
    def k(dst_hbm, z_hbm, ones_hbm, out_hbm, dst_v, ones_v, acc):
        cid = lax.axis_index("c")
        sid = lax.axis_index("s")
        row0 = pl.multiple_of(sid * RPT, 8)
        g0 = pl.multiple_of((cid * NS + sid) * GP, 8)
        pltpu.sync_copy(z_hbm.at[pl.ds(row0, RPT)],
                        acc.at[pl.ds(row0, RPT)])
        pltpu.sync_copy(dst_hbm.at[pl.ds(g0, GP)], dst_v)
        pltpu.sync_copy(ones_hbm, ones_v)
        plsc.subcore_barrier()

        def body(j, carry):
            pltpu.sync_copy(ones_v, acc.at[dst_v.at[j]], add=True)
            return carry

        lax.fori_loop(0, GP, body, 0)
        plsc.subcore_barrier()

        @pl.when(sid != NS - 1)
        def _():
            pltpu.sync_copy(acc.at[pl.ds(row0, RPT)],
                            out_hbm.at[cid, pl.ds(row0, RPT)])

        @pl.when(sid == NS - 1)
        def _():
            pltpu.sync_copy(acc.at[pl.ds(row0, RPT_LAST)],
                            out_hbm.at[cid, pl.ds(row0, RPT_LAST)])

    return k(dst_groups, zeros, ones)


def _sc_edge_agg(src_groups, dst_groups, y, zeros, d, nb=2, notc=False):
    mesh = plsc.VectorSubcoreMesh(core_axis_name="c", subcore_axis_name="s")

    hgp = GP // 2

    @functools.partial(
        pl.kernel,
        mesh=mesh,
        compiler_params=(pltpu.CompilerParams(use_tc_tiling_on_sc=False)
                         if notc else None),
        out_type=jax.ShapeDtypeStruct((NC, N, d), jnp.float32),
        scratch_types=[
            pltpu.VMEM((GP, G), jnp.int32),
            pltpu.VMEM((hgp, G), jnp.int32),
            [pltpu.VMEM((G, d), jnp.float32)] * nb,
            pltpu.VMEM_SHARED((N_PAD, d), jnp.float32),
            [pltpu.SemaphoreType.DMA] * nb,
            [pltpu.SemaphoreType.DMA] * nb,
        ],
    )
    def k(src_hbm, dst_hbm, y_hbm, z_hbm, out_hbm, src_v, dst_v, rows_v, acc,
          sems, ssems):
        cid = lax.axis_index("c")
        sid = lax.axis_index("s")
        row0 = pl.multiple_of(sid * RPT, 8)
        g0 = pl.multiple_of((cid * NS + sid) * GP, 8)
        pltpu.sync_copy(z_hbm.at[pl.ds(row0, RPT)],
                        acc.at[pl.ds(row0, RPT)])
        pltpu.sync_copy(src_hbm.at[pl.ds(g0, GP)], src_v)
        pltpu.sync_copy(dst_hbm.at[pl.ds(g0, hgp)], dst_v)
        plsc.subcore_barrier()

        for b in range(nb):
            pltpu.async_copy(y_hbm.at[src_v.at[b]], rows_v[b], sems[b])

        for h in range(2):
            if h == 1:
                pltpu.sync_copy(dst_hbm.at[pl.ds(g0 + hgp, hgp)], dst_v)

            def body(j2, carry):
                j = h * hgp + j2 * nb
                for b in range(nb):
                    pltpu.make_async_copy(
                        y_hbm.at[src_v.at[j + b]], rows_v[b], sems[b]).wait()
                    pltpu.async_copy(rows_v[b],
                                     acc.at[dst_v.at[j2 * nb + b]], ssems[b],
                                     add=True)
                for b in range(nb):
                    g = j + b
                    pltpu.make_async_copy(
                        rows_v[b], acc.at[dst_v.at[j2 * nb + b]],
                        ssems[b]).wait()

                    @pl.when(g + nb < GP)
                    def _():
                        pltpu.async_copy(
                            y_hbm.at[src_v.at[g + nb]], rows_v[b], sems[b])
                return carry

            lax.fori_loop(0, hgp // nb, body, 0)
        plsc.subcore_barrier()

        @pl.when(sid != NS - 1)
        def _():
            pltpu.sync_copy(acc.at[pl.ds(row0, RPT)],
                            out_hbm.at[cid, pl.ds(row0, RPT)])

        @pl.when(sid == NS - 1)
        def _():
            pltpu.sync_copy(acc.at[pl.ds(row0, RPT_LAST)],
                            out_hbm.at[cid, pl.ds(row0, RPT_LAST)])

    return k(src_groups, dst_groups, y, zeros)


_RB = 1000


def _tc_matmul(x, w):
    def body(x_ref, w_ref, o_ref):
        o_ref[...] = jnp.dot(x_ref[...], w_ref[...],
                             preferred_element_type=jnp.float32)

    return pl.pallas_call(
        body,
        grid=(N // _RB,),
        in_specs=[
            pl.BlockSpec((_RB, x.shape[1]), lambda i: (i, 0)),
            pl.BlockSpec(w.shape, lambda i: (0, 0)),
        ],
        out_specs=pl.BlockSpec((_RB, w.shape[1]), lambda i: (i, 0)),
        out_shape=jax.ShapeDtypeStruct((N, w.shape[1]), jnp.float32),
    )(x, w)


def _tc_scale1(deg_parts, xw1):
    def body(dp_ref, xw_ref, y_ref, deg_ref):
        deg = dp_ref[0, :, 0:1] + dp_ref[1, :, 0:1] + 1.0
        y_ref[...] = xw_ref[...] * lax.rsqrt(deg)
        deg_ref[...] = deg

    return pl.pallas_call(
        body,
        grid=(N // _RB,),
        in_specs=[
            pl.BlockSpec((NC, _RB, HID_D), lambda i: (0, i, 0)),
            pl.BlockSpec((_RB, HID_D), lambda i: (i, 0)),
        ],
        out_specs=[
            pl.BlockSpec((_RB, HID_D), lambda i: (i, 0)),
            pl.BlockSpec((_RB, 1), lambda i: (i, 0)),
        ],
        out_shape=[
            jax.ShapeDtypeStruct((N, HID_D), jnp.float32),
            jax.ShapeDtypeStruct((N, 1), jnp.float32),
        ],
    )(deg_parts, xw1)


def _tc_mid(agg1, deg, y1, b1, w2):
    def body(a_ref, deg_ref, y_ref, b_ref, w_ref, y2_ref):
        dis = lax.rsqrt(deg_ref[...])
        h = jnp.maximum((a_ref[0] + a_ref[1] + y_ref[...]) * dis + b_ref[...],
                        0.0)
        y2_ref[...] = jnp.dot(h, w_ref[...],
                              preferred_element_type=jnp.float32) * dis

    return pl.pallas_call(
        body,
        grid=(N // _RB,),
        in_specs=[
            pl.BlockSpec((NC, _RB, HID_D), lambda i: (0, i, 0)),
            pl.BlockSpec((_RB, 1), lambda i: (i, 0)),
            pl.BlockSpec((_RB, HID_D), lambda i: (i, 0)),
            pl.BlockSpec((1, HID_D), lambda i: (0, 0)),
            pl.BlockSpec((HID_D, OUT_D), lambda i: (0, 0)),
        ],
        out_specs=pl.BlockSpec((_RB, OUT_D), lambda i: (i, 0)),
        out_shape=jax.ShapeDtypeStruct((N, OUT_D), jnp.float32),
    )(agg1, deg, y1, b1, w2)


def _tc_final(agg2, deg, y2, b2):
    def body(a_ref, deg_ref, y_ref, b_ref, o_ref):
        dis = lax.rsqrt(deg_ref[...])
        o_ref[...] = (a_ref[0] + a_ref[1] + y_ref[...]) * dis + b_ref[...]

    return pl.pallas_call(
        body,
        grid=(N // _RB,),
        in_specs=[
            pl.BlockSpec((NC, _RB, OUT_D), lambda i: (0, i, 0)),
            pl.BlockSpec((_RB, 1), lambda i: (i, 0)),
            pl.BlockSpec((_RB, OUT_D), lambda i: (i, 0)),
            pl.BlockSpec((1, OUT_D), lambda i: (0, 0)),
        ],
        out_specs=pl.BlockSpec((_RB, OUT_D), lambda i: (i, 0)),
        out_shape=jax.ShapeDtypeStruct((N, OUT_D), jnp.float32),
    )(agg2, deg, y2, b2)


def kernel(x, edge_index, W1, b1, W2, b2):
    ei = edge_index.astype(jnp.int32)
    pad = E_PAD - E
    src_groups = jnp.concatenate(
        [ei[0], jnp.zeros((pad,), jnp.int32)]).reshape(GROUPS_PAD, G)
    pad_dst = DUMMY_DST0 + jnp.arange(pad, dtype=jnp.int32) % N_DUMMY
    dst_groups = jnp.concatenate(
        [ei[1], pad_dst]).reshape(GROUPS_PAD, G)

    zeros128 = jnp.zeros((N_PAD, HID_D), jnp.float32)
    zeros64 = jnp.zeros((N_PAD, OUT_D), jnp.float32)
    ones128 = jnp.ones((G, HID_D), jnp.float32)

    deg_parts = _sc_degree(dst_groups, zeros128, ones128)
    xw1 = _tc_matmul(x, W1)
    y1, deg = _tc_scale1(deg_parts, xw1)
    agg1 = _sc_edge_agg(src_groups, dst_groups, y1, zeros128, HID_D)
    y2 = _tc_mid(agg1, deg, y1, b1.reshape(1, HID_D), W2)
    agg2 = _sc_edge_agg(src_groups, dst_groups, y2, zeros64, OUT_D,
                        nb=4, notc=True)
    return _tc_final(agg2, deg, y2, b2.reshape(1, OUT_D))


# speedup vs baseline: 1.0137x; 1.0137x over previous
import functools

import jax
import jax.numpy as jnp
from jax import lax
from jax.experimental import pallas as pl
from jax.experimental.pallas import tpu as pltpu
from jax.experimental.pallas import tpu_sc as plsc

N = 10000
E = 320000
IN_D = 128
HID_D = 128
OUT_D = 64

NC = 2
NS = 16
NW = NC * NS

G = 128
GP = 80
GROUPS_PAD = NW * GP
E_PAD = GROUPS_PAD * G

N_PAD = 10112
RPT = N_PAD // NS
RPT_LAST = N - (NS - 1) * RPT
DUMMY_DST0 = N
N_DUMMY = N_PAD - N


def _sc_degree(dst_groups, zeros, ones):
    mesh = plsc.VectorSubcoreMesh(core_axis_name="c", subcore_axis_name="s")

    @functools.partial(
        pl.kernel,
        mesh=mesh,
        out_type=jax.ShapeDtypeStruct((NC, N, HID_D), jnp.float32),
        scratch_types=[
            pltpu.VMEM((GP, G), jnp.int32),
            pltpu.VMEM((G, HID_D), jnp.float32),
            pltpu.VMEM_SHARED((N_PAD, HID_D), jnp.float32),
        ],
    )
    def k(dst_hbm, z_hbm, ones_hbm, out_hbm, dst_v, ones_v, acc):
        cid = lax.axis_index("c")
        sid = lax.axis_index("s")
        row0 = pl.multiple_of(sid * RPT, 8)
        g0 = pl.multiple_of((cid * NS + sid) * GP, 8)
        pltpu.sync_copy(z_hbm.at[pl.ds(row0, RPT)],
                        acc.at[pl.ds(row0, RPT)])
        pltpu.sync_copy(dst_hbm.at[pl.ds(g0, GP)], dst_v)
        pltpu.sync_copy(ones_hbm, ones_v)
        plsc.subcore_barrier()

        def body(j, carry):
            pltpu.sync_copy(ones_v, acc.at[dst_v.at[j]], add=True)
            return carry

        lax.fori_loop(0, GP, body, 0)
        plsc.subcore_barrier()

        @pl.when(sid != NS - 1)
        def _():
            pltpu.sync_copy(acc.at[pl.ds(row0, RPT)],
                            out_hbm.at[cid, pl.ds(row0, RPT)])

        @pl.when(sid == NS - 1)
        def _():
            pltpu.sync_copy(acc.at[pl.ds(row0, RPT_LAST)],
                            out_hbm.at[cid, pl.ds(row0, RPT_LAST)])

    return k(dst_groups, zeros, ones)


def _sc_edge_agg(src_groups, dst_groups, y, zeros, d, nb=2, notc=False):
    mesh = plsc.VectorSubcoreMesh(core_axis_name="c", subcore_axis_name="s")

    hgp = GP // 2
    hg = G // 2

    @functools.partial(
        pl.kernel,
        mesh=mesh,
        compiler_params=(pltpu.CompilerParams(use_tc_tiling_on_sc=False)
                         if notc else None),
        out_type=jax.ShapeDtypeStruct((NC, N, d), jnp.float32),
        scratch_types=[
            pltpu.VMEM((GP, G), jnp.int32),
            pltpu.VMEM((hgp, G), jnp.int32),
            [pltpu.VMEM((hg, d), jnp.float32)] * (2 * nb),
            pltpu.VMEM_SHARED((N_PAD, d), jnp.float32),
            [pltpu.SemaphoreType.DMA] * (2 * nb),
        ],
    )
    def k(src_hbm, dst_hbm, y_hbm, z_hbm, out_hbm, src_v, dst_v, rows_v, acc,
          sems):
        cid = lax.axis_index("c")
        sid = lax.axis_index("s")
        row0 = pl.multiple_of(sid * RPT, 8)
        g0 = pl.multiple_of((cid * NS + sid) * GP, 8)
        pltpu.sync_copy(z_hbm.at[pl.ds(row0, RPT)],
                        acc.at[pl.ds(row0, RPT)])
        pltpu.sync_copy(src_hbm.at[pl.ds(g0, GP)], src_v)
        pltpu.sync_copy(dst_hbm.at[pl.ds(g0, hgp)], dst_v)
        plsc.subcore_barrier()

        for b in range(nb):
            for s2 in range(2):
                pltpu.async_copy(
                    y_hbm.at[src_v.at[b, pl.ds(s2 * hg, hg)]],
                    rows_v[2 * b + s2], sems[2 * b + s2])

        for h in range(2):
            if h == 1:
                pltpu.sync_copy(dst_hbm.at[pl.ds(g0 + hgp, hgp)], dst_v)

            def body(j2, carry):
                j = h * hgp + j2 * nb
                for b in range(nb):
                    g = j + b
                    for s2 in range(2):
                        pltpu.make_async_copy(
                            y_hbm.at[src_v.at[g, pl.ds(s2 * hg, hg)]],
                            rows_v[2 * b + s2], sems[2 * b + s2]).wait()
                        pltpu.sync_copy(
                            rows_v[2 * b + s2],
                            acc.at[dst_v.at[j2 * nb + b, pl.ds(s2 * hg, hg)]],
                            add=True)

                        @pl.when(g + nb < GP)
                        def _():
                            pltpu.async_copy(
                                y_hbm.at[src_v.at[g + nb, pl.ds(s2 * hg, hg)]],
                                rows_v[2 * b + s2], sems[2 * b + s2])
                return carry

            lax.fori_loop(0, hgp // nb, body, 0)
        plsc.subcore_barrier()

        @pl.when(sid != NS - 1)
        def _():
            pltpu.sync_copy(acc.at[pl.ds(row0, RPT)],
                            out_hbm.at[cid, pl.ds(row0, RPT)])

        @pl.when(sid == NS - 1)
        def _():
            pltpu.sync_copy(acc.at[pl.ds(row0, RPT_LAST)],
                            out_hbm.at[cid, pl.ds(row0, RPT_LAST)])

    return k(src_groups, dst_groups, y, zeros)


_RB = 1000


def _tc_matmul(x, w):
    def body(x_ref, w_ref, o_ref):
        o_ref[...] = jnp.dot(x_ref[...], w_ref[...],
                             preferred_element_type=jnp.float32)

    return pl.pallas_call(
        body,
        grid=(N // _RB,),
        in_specs=[
            pl.BlockSpec((_RB, x.shape[1]), lambda i: (i, 0)),
            pl.BlockSpec(w.shape, lambda i: (0, 0)),
        ],
        out_specs=pl.BlockSpec((_RB, w.shape[1]), lambda i: (i, 0)),
        out_shape=jax.ShapeDtypeStruct((N, w.shape[1]), jnp.float32),
    )(x, w)


def _tc_scale1(deg_parts, xw1):
    def body(dp_ref, xw_ref, y_ref, deg_ref):
        deg = dp_ref[0, :, 0:1] + dp_ref[1, :, 0:1] + 1.0
        y_ref[...] = xw_ref[...] * lax.rsqrt(deg)
        deg_ref[...] = deg

    return pl.pallas_call(
        body,
        grid=(N // _RB,),
        in_specs=[
            pl.BlockSpec((NC, _RB, HID_D), lambda i: (0, i, 0)),
            pl.BlockSpec((_RB, HID_D), lambda i: (i, 0)),
        ],
        out_specs=[
            pl.BlockSpec((_RB, HID_D), lambda i: (i, 0)),
            pl.BlockSpec((_RB, 1), lambda i: (i, 0)),
        ],
        out_shape=[
            jax.ShapeDtypeStruct((N, HID_D), jnp.float32),
            jax.ShapeDtypeStruct((N, 1), jnp.float32),
        ],
    )(deg_parts, xw1)


def _tc_mid(agg1, deg, y1, b1, w2):
    def body(a_ref, deg_ref, y_ref, b_ref, w_ref, y2_ref):
        dis = lax.rsqrt(deg_ref[...])
        h = jnp.maximum((a_ref[0] + a_ref[1] + y_ref[...]) * dis + b_ref[...],
                        0.0)
        y2_ref[...] = jnp.dot(h, w_ref[...],
                              preferred_element_type=jnp.float32) * dis

    return pl.pallas_call(
        body,
        grid=(N // _RB,),
        in_specs=[
            pl.BlockSpec((NC, _RB, HID_D), lambda i: (0, i, 0)),
            pl.BlockSpec((_RB, 1), lambda i: (i, 0)),
            pl.BlockSpec((_RB, HID_D), lambda i: (i, 0)),
            pl.BlockSpec((1, HID_D), lambda i: (0, 0)),
            pl.BlockSpec((HID_D, OUT_D), lambda i: (0, 0)),
        ],
        out_specs=pl.BlockSpec((_RB, OUT_D), lambda i: (i, 0)),
        out_shape=jax.ShapeDtypeStruct((N, OUT_D), jnp.float32),
    )(agg1, deg, y1, b1, w2)


def _tc_final(agg2, deg, y2, b2):
    def body(a_ref, deg_ref, y_ref, b_ref, o_ref):
        dis = lax.rsqrt(deg_ref[...])
        o_ref[...] = (a_ref[0] + a_ref[1] + y_ref[...]) * dis + b_ref[...]

    return pl.pallas_call(
        body,
        grid=(N // _RB,),
        in_specs=[
            pl.BlockSpec((NC, _RB, OUT_D), lambda i: (0, i, 0)),
            pl.BlockSpec((_RB, 1), lambda i: (i, 0)),
            pl.BlockSpec((_RB, OUT_D), lambda i: (i, 0)),
            pl.BlockSpec((1, OUT_D), lambda i: (0, 0)),
        ],
        out_specs=pl.BlockSpec((_RB, OUT_D), lambda i: (i, 0)),
        out_shape=jax.ShapeDtypeStruct((N, OUT_D), jnp.float32),
    )(agg2, deg, y2, b2)


def kernel(x, edge_index, W1, b1, W2, b2):
    ei = edge_index.astype(jnp.int32)
    pad = E_PAD - E
    src_groups = jnp.concatenate(
        [ei[0], jnp.zeros((pad,), jnp.int32)]).reshape(GROUPS_PAD, G)
    pad_dst = DUMMY_DST0 + jnp.arange(pad, dtype=jnp.int32) % N_DUMMY
    dst_groups = jnp.concatenate(
        [ei[1], pad_dst]).reshape(GROUPS_PAD, G)

    zeros128 = jnp.zeros((N_PAD, HID_D), jnp.float32)
    zeros64 = jnp.zeros((N_PAD, OUT_D), jnp.float32)
    ones128 = jnp.ones((G, HID_D), jnp.float32)

    deg_parts = _sc_degree(dst_groups, zeros128, ones128)
    xw1 = _tc_matmul(x, W1)
    y1, deg = _tc_scale1(deg_parts, xw1)
    agg1 = _sc_edge_agg(src_groups, dst_groups, y1, zeros128, HID_D)
    y2 = _tc_mid(agg1, deg, y1, b1.reshape(1, HID_D), W2)
    agg2 = _sc_edge_agg(src_groups, dst_groups, y2, zeros64, OUT_D,
                        nb=4, notc=True)
    return _tc_final(agg2, deg, y2, b2.reshape(1, OUT_D))


# speedup vs baseline: 3.0147x; 2.9740x over previous
import functools

import jax
import jax.numpy as jnp
from jax import lax
from jax.experimental import pallas as pl
from jax.experimental.pallas import tpu as pltpu
from jax.experimental.pallas import tpu_sc as plsc

N = 10000
E = 320000
IN_D = 128
HID_D = 128
OUT_D = 64

NC = 2
NS = 16
NW = NC * NS

G = 128
GP = 80
GROUPS_PAD = NW * GP
E_PAD = GROUPS_PAD * G

N_PAD = 10112
RPT = N_PAD // NS
RPT_LAST = N - (NS - 1) * RPT
DUMMY_DST0 = N
N_DUMMY = N_PAD - N


def _sc_degree(dst_groups, zeros, ones):
    mesh = plsc.VectorSubcoreMesh(core_axis_name="c", subcore_axis_name="s")

    @functools.partial(
        pl.kernel,
        mesh=mesh,
        out_type=jax.ShapeDtypeStruct((NC, N, HID_D), jnp.float32),
        scratch_types=[
            pltpu.VMEM((GP, G), jnp.int32),
            pltpu.VMEM((G, HID_D), jnp.float32),
            pltpu.VMEM_SHARED((N_PAD, HID_D), jnp.float32),
        ],
    )
    def k(dst_hbm, z_hbm, ones_hbm, out_hbm, dst_v, ones_v, acc):
        cid = lax.axis_index("c")
        sid = lax.axis_index("s")
        row0 = pl.multiple_of(sid * RPT, 8)
        g0 = pl.multiple_of((cid * NS + sid) * GP, 8)
        pltpu.sync_copy(z_hbm.at[pl.ds(row0, RPT)],
                        acc.at[pl.ds(row0, RPT)])
        pltpu.sync_copy(dst_hbm.at[pl.ds(g0, GP)], dst_v)
        pltpu.sync_copy(ones_hbm, ones_v)
        plsc.subcore_barrier()

        def body(j, carry):
            pltpu.sync_copy(ones_v, acc.at[dst_v.at[j]], add=True)
            return carry

        lax.fori_loop(0, GP, body, 0)
        plsc.subcore_barrier()

        @pl.when(sid != NS - 1)
        def _():
            pltpu.sync_copy(acc.at[pl.ds(row0, RPT)],
                            out_hbm.at[cid, pl.ds(row0, RPT)])

        @pl.when(sid == NS - 1)
        def _():
            pltpu.sync_copy(acc.at[pl.ds(row0, RPT_LAST)],
                            out_hbm.at[cid, pl.ds(row0, RPT_LAST)])

    return k(dst_groups, zeros, ones)


def _sc_edge_agg(src_groups, dst_groups, y, zeros, d, nb=2, notc=False):
    mesh = plsc.VectorSubcoreMesh(core_axis_name="c", subcore_axis_name="s")

    hgp = GP // 2
    hg = G // 2

    @functools.partial(
        pl.kernel,
        mesh=mesh,
        compiler_params=(pltpu.CompilerParams(use_tc_tiling_on_sc=False)
                         if notc else None),
        out_type=jax.ShapeDtypeStruct((NC, N, d), jnp.float32),
        scratch_types=[
            pltpu.VMEM((GP, G), jnp.int32),
            pltpu.VMEM((hgp, G), jnp.int32),
            [pltpu.VMEM((hg, d), jnp.float32)] * (2 * nb),
            pltpu.VMEM_SHARED((N_PAD, d), jnp.float32),
            [pltpu.SemaphoreType.DMA] * (2 * nb),
        ],
    )
    def k(src_hbm, dst_hbm, y_hbm, z_hbm, out_hbm, src_v, dst_v, rows_v, acc,
          sems):
        cid = lax.axis_index("c")
        sid = lax.axis_index("s")
        row0 = pl.multiple_of(sid * RPT, 8)
        g0 = pl.multiple_of((cid * NS + sid) * GP, 8)
        pltpu.sync_copy(z_hbm.at[pl.ds(row0, RPT)],
                        acc.at[pl.ds(row0, RPT)])
        pltpu.sync_copy(src_hbm.at[pl.ds(g0, GP)], src_v)
        pltpu.sync_copy(dst_hbm.at[pl.ds(g0, hgp)], dst_v)
        plsc.subcore_barrier()

        for b in range(nb):
            for s2 in range(2):
                pltpu.async_copy(
                    y_hbm.at[src_v.at[b, pl.ds(s2 * hg, hg)]],
                    rows_v[2 * b + s2], sems[2 * b + s2])

        for h in range(2):
            if h == 1:
                pltpu.sync_copy(dst_hbm.at[pl.ds(g0 + hgp, hgp)], dst_v)

            def body(j2, carry):
                j = h * hgp + j2 * nb
                for b in range(nb):
                    g = j + b
                    for s2 in range(2):
                        pltpu.make_async_copy(
                            y_hbm.at[src_v.at[g, pl.ds(s2 * hg, hg)]],
                            rows_v[2 * b + s2], sems[2 * b + s2]).wait()
                        pltpu.sync_copy(
                            rows_v[2 * b + s2],
                            acc.at[dst_v.at[j2 * nb + b, pl.ds(s2 * hg, hg)]],
                            add=True)

                        @pl.when(g + nb < GP)
                        def _():
                            pltpu.async_copy(
                                y_hbm.at[src_v.at[g + nb, pl.ds(s2 * hg, hg)]],
                                rows_v[2 * b + s2], sems[2 * b + s2])
                return carry

            lax.fori_loop(0, hgp // nb, body, 0)
        plsc.subcore_barrier()

        @pl.when(sid != NS - 1)
        def _():
            pltpu.sync_copy(acc.at[pl.ds(row0, RPT)],
                            out_hbm.at[cid, pl.ds(row0, RPT)])

        @pl.when(sid == NS - 1)
        def _():
            pltpu.sync_copy(acc.at[pl.ds(row0, RPT_LAST)],
                            out_hbm.at[cid, pl.ds(row0, RPT_LAST)])

    return k(src_groups, dst_groups, y, zeros)


_RB = 1000


def _tc_matmul(x, w):
    def body(x_ref, w_ref, o_ref):
        o_ref[...] = jnp.dot(x_ref[...], w_ref[...],
                             preferred_element_type=jnp.float32)

    return pl.pallas_call(
        body,
        grid=(N // _RB,),
        in_specs=[
            pl.BlockSpec((_RB, x.shape[1]), lambda i: (i, 0)),
            pl.BlockSpec(w.shape, lambda i: (0, 0)),
        ],
        out_specs=pl.BlockSpec((_RB, w.shape[1]), lambda i: (i, 0)),
        out_shape=jax.ShapeDtypeStruct((N, w.shape[1]), jnp.float32),
    )(x, w)


def _tc_scale1(deg_parts, xw1):
    def body(dp_ref, xw_ref, y_ref, deg_ref):
        deg = dp_ref[0, :, 0:1] + dp_ref[1, :, 0:1] + 1.0
        y_ref[...] = xw_ref[...] * lax.rsqrt(deg)
        deg_ref[...] = deg

    return pl.pallas_call(
        body,
        grid=(N // _RB,),
        in_specs=[
            pl.BlockSpec((NC, _RB, HID_D), lambda i: (0, i, 0)),
            pl.BlockSpec((_RB, HID_D), lambda i: (i, 0)),
        ],
        out_specs=[
            pl.BlockSpec((_RB, HID_D), lambda i: (i, 0)),
            pl.BlockSpec((_RB, 1), lambda i: (i, 0)),
        ],
        out_shape=[
            jax.ShapeDtypeStruct((N, HID_D), jnp.float32),
            jax.ShapeDtypeStruct((N, 1), jnp.float32),
        ],
    )(deg_parts, xw1)


def _tc_mid(agg1, deg, y1, b1, w2):
    def body(a_ref, deg_ref, y_ref, b_ref, w_ref, y2_ref):
        dis = lax.rsqrt(deg_ref[...])
        h = jnp.maximum((a_ref[0] + a_ref[1] + y_ref[...]) * dis + b_ref[...],
                        0.0)
        y2_ref[...] = jnp.dot(h, w_ref[...],
                              preferred_element_type=jnp.float32) * dis

    return pl.pallas_call(
        body,
        grid=(N // _RB,),
        in_specs=[
            pl.BlockSpec((NC, _RB, HID_D), lambda i: (0, i, 0)),
            pl.BlockSpec((_RB, 1), lambda i: (i, 0)),
            pl.BlockSpec((_RB, HID_D), lambda i: (i, 0)),
            pl.BlockSpec((1, HID_D), lambda i: (0, 0)),
            pl.BlockSpec((HID_D, OUT_D), lambda i: (0, 0)),
        ],
        out_specs=pl.BlockSpec((_RB, OUT_D), lambda i: (i, 0)),
        out_shape=jax.ShapeDtypeStruct((N, OUT_D), jnp.float32),
    )(agg1, deg, y1, b1, w2)


def _tc_final(agg2, deg, y2, b2):
    def body(a_ref, deg_ref, y_ref, b_ref, o_ref):
        dis = lax.rsqrt(deg_ref[...])
        o_ref[...] = (a_ref[0] + a_ref[1] + y_ref[...]) * dis + b_ref[...]

    return pl.pallas_call(
        body,
        grid=(N // _RB,),
        in_specs=[
            pl.BlockSpec((NC, _RB, OUT_D), lambda i: (0, i, 0)),
            pl.BlockSpec((_RB, 1), lambda i: (i, 0)),
            pl.BlockSpec((_RB, OUT_D), lambda i: (i, 0)),
            pl.BlockSpec((1, OUT_D), lambda i: (0, 0)),
        ],
        out_specs=pl.BlockSpec((_RB, OUT_D), lambda i: (i, 0)),
        out_shape=jax.ShapeDtypeStruct((N, OUT_D), jnp.float32),
    )(agg2, deg, y2, b2)


def kernel(x, edge_index, W1, b1, W2, b2):
    ei = edge_index.astype(jnp.int32)
    pad = E_PAD - E
    pad_src = jnp.arange(pad, dtype=jnp.int32) * 131 % N
    src_groups = jnp.concatenate(
        [ei[0], pad_src]).reshape(GROUPS_PAD, G)
    pad_dst = DUMMY_DST0 + jnp.arange(pad, dtype=jnp.int32) % N_DUMMY
    dst_groups = jnp.concatenate(
        [ei[1], pad_dst]).reshape(GROUPS_PAD, G)

    zeros128 = jnp.zeros((N_PAD, HID_D), jnp.float32)
    zeros64 = jnp.zeros((N_PAD, OUT_D), jnp.float32)
    ones128 = jnp.ones((G, HID_D), jnp.float32)

    deg_parts = _sc_degree(dst_groups, zeros128, ones128)
    xw1 = _tc_matmul(x, W1)
    y1, deg = _tc_scale1(deg_parts, xw1)
    agg1 = _sc_edge_agg(src_groups, dst_groups, y1, zeros128, HID_D)
    y2 = _tc_mid(agg1, deg, y1, b1.reshape(1, HID_D), W2)
    agg2 = _sc_edge_agg(src_groups, dst_groups, y2, zeros64, OUT_D,
                        nb=4, notc=True)
    return _tc_final(agg2, deg, y2, b2.reshape(1, OUT_D))


# speedup vs baseline: 3.2646x; 1.0829x over previous
import functools

import jax
import jax.numpy as jnp
from jax import lax
from jax.experimental import pallas as pl
from jax.experimental.pallas import tpu as pltpu
from jax.experimental.pallas import tpu_sc as plsc

N = 10000
E = 320000
IN_D = 128
HID_D = 128
OUT_D = 64

NC = 2
NS = 16
NW = NC * NS

G = 128
GP = 80
GROUPS_PAD = NW * GP
E_PAD = GROUPS_PAD * G

N_PAD = 10112
RPT = N_PAD // NS
RPT_LAST = N - (NS - 1) * RPT
DUMMY_DST0 = N
N_DUMMY = N_PAD - N


def _sc_degree(dst_groups, zeros, ones):
    mesh = plsc.VectorSubcoreMesh(core_axis_name="c", subcore_axis_name="s")

    @functools.partial(
        pl.kernel,
        mesh=mesh,
        compiler_params=pltpu.CompilerParams(use_tc_tiling_on_sc=False),
        out_type=jax.ShapeDtypeStruct((NC, N, OUT_D), jnp.float32),
        scratch_types=[
            pltpu.VMEM((GP, G), jnp.int32),
            pltpu.VMEM((G, OUT_D), jnp.float32),
            pltpu.VMEM_SHARED((N_PAD, OUT_D), jnp.float32),
        ],
    )
    def k(dst_hbm, z_hbm, ones_hbm, out_hbm, dst_v, ones_v, acc):
        cid = lax.axis_index("c")
        sid = lax.axis_index("s")
        row0 = pl.multiple_of(sid * RPT, 8)
        g0 = pl.multiple_of((cid * NS + sid) * GP, 8)
        pltpu.sync_copy(z_hbm.at[pl.ds(row0, RPT)],
                        acc.at[pl.ds(row0, RPT)])
        pltpu.sync_copy(dst_hbm.at[pl.ds(g0, GP)], dst_v)
        pltpu.sync_copy(ones_hbm, ones_v)
        plsc.subcore_barrier()

        def body(j, carry):
            pltpu.sync_copy(ones_v, acc.at[dst_v.at[j]], add=True)
            return carry

        lax.fori_loop(0, GP, body, 0)
        plsc.subcore_barrier()

        @pl.when(sid != NS - 1)
        def _():
            pltpu.sync_copy(acc.at[pl.ds(row0, RPT)],
                            out_hbm.at[cid, pl.ds(row0, RPT)])

        @pl.when(sid == NS - 1)
        def _():
            pltpu.sync_copy(acc.at[pl.ds(row0, RPT_LAST)],
                            out_hbm.at[cid, pl.ds(row0, RPT_LAST)])

    return k(dst_groups, zeros, ones)


def _sc_edge_agg(src_groups, dst_groups, y, zeros, d, nb=2, notc=False):
    mesh = plsc.VectorSubcoreMesh(core_axis_name="c", subcore_axis_name="s")

    hgp = GP // 2
    hg = G // 2

    @functools.partial(
        pl.kernel,
        mesh=mesh,
        compiler_params=(pltpu.CompilerParams(use_tc_tiling_on_sc=False)
                         if notc else None),
        out_type=jax.ShapeDtypeStruct((NC, N, d), jnp.float32),
        scratch_types=[
            pltpu.VMEM((GP, G), jnp.int32),
            pltpu.VMEM((hgp, G), jnp.int32),
            [pltpu.VMEM((hg, d), jnp.float32)] * (2 * nb),
            pltpu.VMEM_SHARED((N_PAD, d), jnp.float32),
            [pltpu.SemaphoreType.DMA] * (2 * nb),
        ],
    )
    def k(src_hbm, dst_hbm, y_hbm, z_hbm, out_hbm, src_v, dst_v, rows_v, acc,
          sems):
        cid = lax.axis_index("c")
        sid = lax.axis_index("s")
        row0 = pl.multiple_of(sid * RPT, 8)
        g0 = pl.multiple_of((cid * NS + sid) * GP, 8)
        pltpu.sync_copy(z_hbm.at[pl.ds(row0, RPT)],
                        acc.at[pl.ds(row0, RPT)])
        pltpu.sync_copy(src_hbm.at[pl.ds(g0, GP)], src_v)
        pltpu.sync_copy(dst_hbm.at[pl.ds(g0, hgp)], dst_v)
        plsc.subcore_barrier()

        for b in range(nb):
            for s2 in range(2):
                pltpu.async_copy(
                    y_hbm.at[src_v.at[b, pl.ds(s2 * hg, hg)]],
                    rows_v[2 * b + s2], sems[2 * b + s2])

        for h in range(2):
            if h == 1:
                pltpu.sync_copy(dst_hbm.at[pl.ds(g0 + hgp, hgp)], dst_v)

            def body(j2, carry):
                j = h * hgp + j2 * nb
                for b in range(nb):
                    g = j + b
                    for s2 in range(2):
                        pltpu.make_async_copy(
                            y_hbm.at[src_v.at[g, pl.ds(s2 * hg, hg)]],
                            rows_v[2 * b + s2], sems[2 * b + s2]).wait()
                        pltpu.sync_copy(
                            rows_v[2 * b + s2],
                            acc.at[dst_v.at[j2 * nb + b, pl.ds(s2 * hg, hg)]],
                            add=True)

                        @pl.when(g + nb < GP)
                        def _():
                            pltpu.async_copy(
                                y_hbm.at[src_v.at[g + nb, pl.ds(s2 * hg, hg)]],
                                rows_v[2 * b + s2], sems[2 * b + s2])
                return carry

            lax.fori_loop(0, hgp // nb, body, 0)
        plsc.subcore_barrier()

        @pl.when(sid != NS - 1)
        def _():
            pltpu.sync_copy(acc.at[pl.ds(row0, RPT)],
                            out_hbm.at[cid, pl.ds(row0, RPT)])

        @pl.when(sid == NS - 1)
        def _():
            pltpu.sync_copy(acc.at[pl.ds(row0, RPT_LAST)],
                            out_hbm.at[cid, pl.ds(row0, RPT_LAST)])

    return k(src_groups, dst_groups, y, zeros)


_RB = 1000


def _tc_matmul(x, w):
    def body(x_ref, w_ref, o_ref):
        o_ref[...] = jnp.dot(x_ref[...], w_ref[...],
                             preferred_element_type=jnp.float32)

    return pl.pallas_call(
        body,
        grid=(N // _RB,),
        in_specs=[
            pl.BlockSpec((_RB, x.shape[1]), lambda i: (i, 0)),
            pl.BlockSpec(w.shape, lambda i: (0, 0)),
        ],
        out_specs=pl.BlockSpec((_RB, w.shape[1]), lambda i: (i, 0)),
        out_shape=jax.ShapeDtypeStruct((N, w.shape[1]), jnp.float32),
    )(x, w)


def _tc_mm_scale(deg_parts, x, w1):
    def body(dp_ref, x_ref, w_ref, y_ref, deg_ref):
        deg = dp_ref[0, :, 0:1] + dp_ref[1, :, 0:1] + 1.0
        xw = jnp.dot(x_ref[...], w_ref[...],
                     preferred_element_type=jnp.float32)
        y_ref[...] = xw * lax.rsqrt(deg)
        deg_ref[...] = deg

    return pl.pallas_call(
        body,
        grid=(N // _RB,),
        in_specs=[
            pl.BlockSpec((NC, _RB, OUT_D), lambda i: (0, i, 0)),
            pl.BlockSpec((_RB, IN_D), lambda i: (i, 0)),
            pl.BlockSpec((IN_D, HID_D), lambda i: (0, 0)),
        ],
        out_specs=[
            pl.BlockSpec((_RB, HID_D), lambda i: (i, 0)),
            pl.BlockSpec((_RB, 1), lambda i: (i, 0)),
        ],
        out_shape=[
            jax.ShapeDtypeStruct((N, HID_D), jnp.float32),
            jax.ShapeDtypeStruct((N, 1), jnp.float32),
        ],
    )(deg_parts, x, w1)


def _tc_mid(agg1, deg, y1, b1, w2):
    def body(a_ref, deg_ref, y_ref, b_ref, w_ref, y2_ref):
        dis = lax.rsqrt(deg_ref[...])
        h = jnp.maximum((a_ref[0] + a_ref[1] + y_ref[...]) * dis + b_ref[...],
                        0.0)
        y2_ref[...] = jnp.dot(h, w_ref[...],
                              preferred_element_type=jnp.float32) * dis

    return pl.pallas_call(
        body,
        grid=(N // _RB,),
        in_specs=[
            pl.BlockSpec((NC, _RB, HID_D), lambda i: (0, i, 0)),
            pl.BlockSpec((_RB, 1), lambda i: (i, 0)),
            pl.BlockSpec((_RB, HID_D), lambda i: (i, 0)),
            pl.BlockSpec((1, HID_D), lambda i: (0, 0)),
            pl.BlockSpec((HID_D, OUT_D), lambda i: (0, 0)),
        ],
        out_specs=pl.BlockSpec((_RB, OUT_D), lambda i: (i, 0)),
        out_shape=jax.ShapeDtypeStruct((N, OUT_D), jnp.float32),
    )(agg1, deg, y1, b1, w2)


def _tc_final(agg2, deg, y2, b2):
    def body(a_ref, deg_ref, y_ref, b_ref, o_ref):
        dis = lax.rsqrt(deg_ref[...])
        o_ref[...] = (a_ref[0] + a_ref[1] + y_ref[...]) * dis + b_ref[...]

    return pl.pallas_call(
        body,
        grid=(N // _RB,),
        in_specs=[
            pl.BlockSpec((NC, _RB, OUT_D), lambda i: (0, i, 0)),
            pl.BlockSpec((_RB, 1), lambda i: (i, 0)),
            pl.BlockSpec((_RB, OUT_D), lambda i: (i, 0)),
            pl.BlockSpec((1, OUT_D), lambda i: (0, 0)),
        ],
        out_specs=pl.BlockSpec((_RB, OUT_D), lambda i: (i, 0)),
        out_shape=jax.ShapeDtypeStruct((N, OUT_D), jnp.float32),
    )(agg2, deg, y2, b2)


def kernel(x, edge_index, W1, b1, W2, b2):
    ei = edge_index.astype(jnp.int32)
    pad = E_PAD - E
    pad_src = jnp.arange(pad, dtype=jnp.int32) * 131 % N
    src_groups = jnp.concatenate(
        [ei[0], pad_src]).reshape(GROUPS_PAD, G)
    pad_dst = DUMMY_DST0 + jnp.arange(pad, dtype=jnp.int32) % N_DUMMY
    dst_groups = jnp.concatenate(
        [ei[1], pad_dst]).reshape(GROUPS_PAD, G)

    zeros128 = jnp.zeros((N_PAD, HID_D), jnp.float32)
    zeros64 = jnp.zeros((N_PAD, OUT_D), jnp.float32)
    ones64 = jnp.ones((G, OUT_D), jnp.float32)

    deg_parts = _sc_degree(dst_groups, zeros64, ones64)
    y1, deg = _tc_mm_scale(deg_parts, x, W1)
    agg1 = _sc_edge_agg(src_groups, dst_groups, y1, zeros128, HID_D)
    y2 = _tc_mid(agg1, deg, y1, b1.reshape(1, HID_D), W2)
    agg2 = _sc_edge_agg(src_groups, dst_groups, y2, zeros64, OUT_D,
                        nb=4, notc=True)
    return _tc_final(agg2, deg, y2, b2.reshape(1, OUT_D))
